# baseline (speedup 1.0000x reference)
import functools

import jax
import jax.numpy as jnp
from jax import lax
from jax.experimental import pallas as pl
from jax.experimental.pallas import tpu as pltpu
from jax.experimental.pallas import tpu_sc as plsc

_VOCAB = 100000
_EMBED_DIM = 64
_BATCH = 4096
_SEQ_LEN = 200

_NUM_WORKERS = 32
_B_PER_WORKER = _BATCH // _NUM_WORKERS
_CHUNK = _SEQ_LEN // 2
_NCHUNK = 2 * _B_PER_WORKER

_mesh = plsc.VectorSubcoreMesh(core_axis_name="c", subcore_axis_name="s")


@functools.partial(
    pl.kernel,
    mesh=_mesh,
    out_type=jax.ShapeDtypeStruct((_BATCH, _SEQ_LEN, _EMBED_DIM), jnp.float32),
    scratch_types=[
        pltpu.VMEM((_NCHUNK, _CHUNK), jnp.int32),
        pltpu.VMEM((_CHUNK, _EMBED_DIM), jnp.float32),
        pltpu.VMEM((_CHUNK, _EMBED_DIM), jnp.float32),
        pltpu.SemaphoreType.DMA,
        pltpu.SemaphoreType.DMA,
    ],
    compiler_params=pltpu.CompilerParams(use_tc_tiling_on_sc=False),
)
def _embedding_gather(idx_hbm, table_hbm, out_hbm, idx_v, rows0, rows1, sem0, sem1):
    wid = lax.axis_index("s") * 2 + lax.axis_index("c")
    pltpu.sync_copy(idx_hbm.at[pl.ds(wid * _NCHUNK, _NCHUNK)], idx_v)
    b0 = wid * _B_PER_WORKER

    pltpu.async_copy(table_hbm.at[idx_v.at[0]], rows0, sem0)

    def body(i, carry):
        b = b0 + i
        pltpu.make_async_copy(table_hbm.at[idx_v.at[2 * i]], rows0, sem0).wait()
        pltpu.async_copy(table_hbm.at[idx_v.at[2 * i + 1]], rows1, sem1)
        pltpu.sync_copy(rows0, out_hbm.at[b, pl.ds(0, _CHUNK)])

        pltpu.make_async_copy(table_hbm.at[idx_v.at[2 * i + 1]], rows1, sem1).wait()

        @pl.when(i + 1 < _B_PER_WORKER)
        def _():
            pltpu.async_copy(table_hbm.at[idx_v.at[2 * i + 2]], rows0, sem0)

        pltpu.sync_copy(rows1, out_hbm.at[b, pl.ds(_CHUNK, _CHUNK)])
        return carry

    lax.fori_loop(0, _B_PER_WORKER, body, 0)


def kernel(indices, embedding_matrix):
    idx = indices.reshape(_BATCH * 2, _CHUNK).astype(jnp.int32)
    return _embedding_gather(idx, embedding_matrix)
